# baseline (speedup 1.0000x reference)
import functools

import jax
import jax.numpy as jnp
from jax.experimental import pallas as pl
from jax.experimental.pallas import tpu as pltpu


def _vq_kernel(z_ref, zsq_ref, c_ref, csq_ref, q_ref, loss_ref,
               run_min, run_q, *, n_tokens, dim):
    k = pl.program_id(0)
    z = z_ref[...]
    c = c_ref[...]
    csq = csq_ref[0, 0, :]
    zsq = zsq_ref[...]

    dots = jax.lax.dot_general(
        z.astype(jnp.bfloat16), c.astype(jnp.bfloat16),
        (((1,), (1,)), ((), ())),
        preferred_element_type=jnp.float32)
    dist = (zsq - 2.0 * dots) + csq[None, :]

    tile_min = jnp.min(dist, axis=1, keepdims=True)
    iota = jax.lax.broadcasted_iota(jnp.int32, dist.shape, 1)
    big = jnp.int32(dist.shape[1])
    tile_arg = jnp.min(jnp.where(dist == tile_min, iota, big),
                       axis=1, keepdims=True)
    onehot = (iota == tile_arg).astype(jnp.float32)
    tile_q = jax.lax.dot_general(
        onehot, c, (((1,), (0,)), ((), ())),
        preferred_element_type=jnp.float32,
        precision=jax.lax.Precision.HIGHEST)

    @pl.when(k == 0)
    def _init():
        run_min[...] = tile_min
        run_q[...] = tile_q

    @pl.when(k > 0)
    def _update():
        better = tile_min < run_min[...]
        run_min[...] = jnp.where(better, tile_min, run_min[...])
        run_q[...] = jnp.where(better, tile_q, run_q[...])

    @pl.when(k == pl.num_programs(0) - 1)
    def _finish():
        q = run_q[...]
        st = z + (q - z)
        q_ref[...] = st
        d = q - z
        loss_ref[...] = jnp.reshape(jnp.sum(d * d) * (1.0 / (n_tokens * dim)),
                                    (1, 1))


def kernel(x, codebook):
    B, D, N = x.shape
    K = codebook.shape[0]
    T = B * N
    tk = 1024
    n_tiles = K // tk

    z = jnp.swapaxes(x, -1, -2).reshape(T, D)
    z_sq = jnp.sum(z * z, axis=-1, keepdims=True)
    c_sq = jnp.sum(codebook * codebook, axis=-1)
    c_sq3 = c_sq.reshape(n_tiles, 1, tk)

    q, loss = pl.pallas_call(
        functools.partial(_vq_kernel, n_tokens=T, dim=D),
        grid=(n_tiles,),
        in_specs=[
            pl.BlockSpec((T, D), lambda k: (0, 0)),
            pl.BlockSpec((T, 1), lambda k: (0, 0)),
            pl.BlockSpec((tk, D), lambda k: (k, 0)),
            pl.BlockSpec((1, 1, tk), lambda k: (k, 0, 0)),
        ],
        out_specs=[
            pl.BlockSpec((T, D), lambda k: (0, 0)),
            pl.BlockSpec((1, 1), lambda k: (0, 0)),
        ],
        out_shape=[
            jax.ShapeDtypeStruct((T, D), jnp.float32),
            jax.ShapeDtypeStruct((1, 1), jnp.float32),
        ],
        scratch_shapes=[
            pltpu.VMEM((T, 1), jnp.float32),
            pltpu.VMEM((T, D), jnp.float32),
        ],
    )(z, z_sq, codebook, c_sq3)

    out = jnp.swapaxes(q.reshape(B, N, D), -1, -2)
    return out, loss[0, 0]
